# Initial kernel scaffold; baseline (speedup 1.0000x reference)
import jax
import jax.numpy as jnp
from jax.experimental import pallas as pl


def kernel(word_embeddings, states, W_state, b_state, W_lang, b_lang, W0, b0, W1, b1, W2, b2, W_pi, b_pi, W_po, b_po, codebook):
    raise NotImplementedError("write your pallas kernel here")


# speedup vs baseline: 1.1544x; 1.1544x over previous
import jax
import jax.numpy as jnp
from jax import lax
from jax.experimental import pallas as pl
from jax.experimental.pallas import tpu as pltpu
from jax.experimental.pallas import tpu_sc as plsc

_HORIZON = 4
_COMMIT_W = 0.25


def _state_matmul_body(x_ref, w_ref, b_ref, o_ref):
    o_ref[...] = (
        jnp.dot(x_ref[...], w_ref[...], preferred_element_type=jnp.float32)
        + b_ref[...][None, :]
    )


def _vq_body(hs_ref, we_ref, w_state_ref, b_state_ref, w_lang_ref, b_lang_ref,
             w0_ref, b0_ref, w1_ref, b1_ref, w2_ref, b2_ref,
             w_pi_ref, b_pi_ref, w_po_ref, b_po_ref, cb_ref,
             idx_ref, table_ref, loss_ref, ent_ref):
    b = pl.program_id(0)
    nt = hs_ref.shape[1]

    @pl.when(b == 0)
    def _init():
        loss_ref[...] = jnp.zeros((1, 1), jnp.float32)
        ent_ref[...] = jnp.zeros((1, 1), jnp.float32)
        table_ref[...] = (
            jnp.dot(cb_ref[...], w_po_ref[...], preferred_element_type=jnp.float32)
            + b_po_ref[...][None, :]
        )

    hs = hs_ref[0]
    se = jnp.dot(hs, w_state_ref[...], preferred_element_type=jnp.float32)
    se = se + b_state_ref[...][None, :]

    we = we_ref[0]
    le = jnp.dot(we, w_lang_ref[...], preferred_element_type=jnp.float32)
    le = le + b_lang_ref[...][None, :]
    le_rep = jnp.broadcast_to(le, (nt, le.shape[1]))

    inp = jnp.concatenate([le_rep, se], axis=-1)
    h = jnp.dot(inp, w0_ref[...], preferred_element_type=jnp.float32) + b0_ref[...][None, :]
    h = jnp.dot(h, w1_ref[...], preferred_element_type=jnp.float32) + b1_ref[...][None, :]
    op = jnp.dot(h, w2_ref[...], preferred_element_type=jnp.float32) + b2_ref[...][None, :]
    x = jnp.dot(op, w_pi_ref[...], preferred_element_type=jnp.float32) + b_pi_ref[...][None, :]

    cb = cb_ref[...]
    xc = lax.dot_general(x, cb, (((1,), (1,)), ((), ())),
                         preferred_element_type=jnp.float32)
    x2 = jnp.sum(x * x, axis=1, keepdims=True)
    c2 = jnp.sum(cb * cb, axis=1)
    d = x2 - 2.0 * xc + c2[None, :]

    idx_ref[0, 0, :] = jnp.argmin(d, axis=1).astype(jnp.int32)

    dmin = jnp.min(d, axis=1)
    cd = w_pi_ref.shape[1]
    loss_scale = _COMMIT_W / (pl.num_programs(0) * nt * cd)
    loss_ref[...] += (jnp.sum(dmin) * loss_scale).reshape(1, 1)

    neg = -d
    m = jnp.max(neg, axis=1, keepdims=True)
    e = jnp.exp(neg - m)
    p = e / jnp.sum(e, axis=1, keepdims=True)
    ent = -jnp.sum(p * jnp.log(p + 1e-9), axis=1)
    ent_ref[...] += (jnp.sum(ent) * (1.0 / (pl.num_programs(0) * nt))).reshape(1, 1)


def _sc_gather(table_hbm, idx_hbm, out_hbm, idx_v, rows_v, sem):
    wid = lax.axis_index("s") * 2 + lax.axis_index("c")
    b_per_w = idx_v.shape[0]
    base = wid * b_per_w
    pltpu.sync_copy(idx_hbm.at[pl.ds(base, b_per_w)], idx_v)
    pltpu.async_copy(table_hbm.at[idx_v], rows_v, sem).wait()
    pltpu.sync_copy(rows_v, out_hbm.at[pl.ds(base, b_per_w)])


def kernel(word_embeddings, states, W_state, b_state, W_lang, b_lang,
           W0, b0, W1, b1, W2, b2, W_pi, b_pi, W_po, b_po, codebook):
    B, T, S = states.shape
    L = word_embeddings.shape[2]
    H = W_state.shape[1]
    D = W2.shape[1]
    CD = W_pi.shape[1]
    K = codebook.shape[0]
    NT = T // _HORIZON

    rows = B * T
    row_tile = 2048
    flat = states.reshape(rows, S)
    ret_state = pl.pallas_call(
        _state_matmul_body,
        grid=(rows // row_tile,),
        in_specs=[
            pl.BlockSpec((row_tile, S), lambda i: (i, 0)),
            pl.BlockSpec((S, H), lambda i: (0, 0)),
            pl.BlockSpec((H,), lambda i: (0,)),
        ],
        out_specs=pl.BlockSpec((row_tile, H), lambda i: (i, 0)),
        out_shape=jax.ShapeDtypeStruct((rows, H), jnp.float32),
    )(flat, W_state, b_state).reshape(B, T, H)

    hs_view = states.reshape(B, NT, _HORIZON * S)
    full = lambda shape: pl.BlockSpec(shape, lambda b: tuple(0 for _ in shape))
    idx3, table, loss11, ent11 = pl.pallas_call(
        _vq_body,
        grid=(B,),
        in_specs=[
            pl.BlockSpec((1, NT, S), lambda b: (b, 0, 0)),
            pl.BlockSpec((1, 1, L), lambda b: (b, 0, 0)),
            full((S, H)), full((H,)),
            full((L, H)), full((H,)),
            full((2 * H, H)), full((H,)),
            full((H, H)), full((H,)),
            full((H, D)), full((D,)),
            full((D, CD)), full((CD,)),
            full((CD, D)), full((D,)),
            full((K, CD)),
        ],
        out_specs=[
            pl.BlockSpec((1, 1, NT), lambda b: (b, 0, 0)),
            full((K, D)),
            full((1, 1)),
            full((1, 1)),
        ],
        out_shape=[
            jax.ShapeDtypeStruct((B, 1, NT), jnp.int32),
            jax.ShapeDtypeStruct((K, D), jnp.float32),
            jax.ShapeDtypeStruct((1, 1), jnp.float32),
            jax.ShapeDtypeStruct((1, 1), jnp.float32),
        ],
    )(hs_view, word_embeddings, W_state, b_state, W_lang, b_lang,
      W0, b0, W1, b1, W2, b2, W_pi, b_pi, W_po, b_po, codebook)

    indices = idx3.reshape(B, NT)

    ntok = B * NT
    n_workers = 32
    b_per_w = ntok // n_workers
    mesh = plsc.VectorSubcoreMesh(core_axis_name="c", subcore_axis_name="s")
    gather = pl.kernel(
        _sc_gather, mesh=mesh,
        out_type=jax.ShapeDtypeStruct((ntok, D), jnp.float32),
        scratch_types=[
            pltpu.VMEM((b_per_w,), jnp.int32),
            pltpu.VMEM((b_per_w, D), jnp.float32),
            pltpu.SemaphoreType.DMA,
        ],
    )
    options = gather(table, indices.reshape(ntok)).reshape(B, NT, D)

    commitment_loss = loss11[0, 0]
    entropies = ent11[0, 0]
    return (options, indices, commitment_loss, entropies, ret_state)


# speedup vs baseline: 1.1811x; 1.0231x over previous
import jax
import jax.numpy as jnp
from jax import lax
from jax.experimental import pallas as pl
from jax.experimental.pallas import tpu as pltpu
from jax.experimental.pallas import tpu_sc as plsc

_HORIZON = 4
_COMMIT_W = 0.25


def _state_matmul_body(x_ref, w_ref, b_ref, o_ref):
    o_ref[...] = (
        jnp.dot(x_ref[...], w_ref[...], preferred_element_type=jnp.float32)
        + b_ref[...][None, :]
    )


def _vq_body(hs_ref, we_ref, w_state_ref, b_state_ref, w_lang_ref, b_lang_ref,
             w0_ref, b0_ref, w1_ref, b1_ref, w2_ref, b2_ref,
             w_pi_ref, b_pi_ref, w_po_ref, b_po_ref, cb_ref,
             idx_ref, table_ref, loss_ref, ent_ref):
    b = pl.program_id(0)
    nt = hs_ref.shape[1]

    @pl.when(b == 0)
    def _init():
        loss_ref[...] = jnp.zeros((1, 1), jnp.float32)
        ent_ref[...] = jnp.zeros((1, 1), jnp.float32)
        table_ref[...] = (
            jnp.dot(cb_ref[...], w_po_ref[...], preferred_element_type=jnp.float32)
            + b_po_ref[...][None, :]
        )

    hs = hs_ref[0]
    se = jnp.dot(hs, w_state_ref[...], preferred_element_type=jnp.float32)
    se = se + b_state_ref[...][None, :]

    we = we_ref[0]
    le = jnp.dot(we, w_lang_ref[...], preferred_element_type=jnp.float32)
    le = le + b_lang_ref[...][None, :]
    le_rep = jnp.broadcast_to(le, (nt, le.shape[1]))

    inp = jnp.concatenate([le_rep, se], axis=-1)
    h = jnp.dot(inp, w0_ref[...], preferred_element_type=jnp.float32) + b0_ref[...][None, :]
    h = jnp.dot(h, w1_ref[...], preferred_element_type=jnp.float32) + b1_ref[...][None, :]
    op = jnp.dot(h, w2_ref[...], preferred_element_type=jnp.float32) + b2_ref[...][None, :]
    x = jnp.dot(op, w_pi_ref[...], preferred_element_type=jnp.float32) + b_pi_ref[...][None, :]

    cb = cb_ref[...]
    xc = lax.dot_general(x, cb, (((1,), (1,)), ((), ())),
                         preferred_element_type=jnp.float32)
    x2 = jnp.sum(x * x, axis=1, keepdims=True)
    c2 = jnp.sum(cb * cb, axis=1)
    d = x2 - 2.0 * xc + c2[None, :]

    idx_ref[0, 0, :] = jnp.argmin(d, axis=1).astype(jnp.int32)

    dmin = jnp.min(d, axis=1)
    cd = w_pi_ref.shape[1]
    loss_scale = _COMMIT_W / (pl.num_programs(0) * nt * cd)
    loss_ref[...] += (jnp.sum(dmin) * loss_scale).reshape(1, 1)

    u = dmin[:, None] - d
    e = jnp.exp(u)
    s = jnp.sum(e, axis=1)
    w = jnp.sum(e * u, axis=1)
    ent = jnp.log(s) - w / s
    ent_ref[...] += (jnp.sum(ent) * (1.0 / (pl.num_programs(0) * nt))).reshape(1, 1)


def _sc_gather(table_hbm, idx_hbm, out_hbm, idx_v, rows_v, sem):
    wid = lax.axis_index("s") * 2 + lax.axis_index("c")
    b_per_w = idx_v.shape[0]
    base = wid * b_per_w
    pltpu.sync_copy(idx_hbm.at[pl.ds(base, b_per_w)], idx_v)
    pltpu.async_copy(table_hbm.at[idx_v], rows_v, sem).wait()
    pltpu.sync_copy(rows_v, out_hbm.at[pl.ds(base, b_per_w)])


def kernel(word_embeddings, states, W_state, b_state, W_lang, b_lang,
           W0, b0, W1, b1, W2, b2, W_pi, b_pi, W_po, b_po, codebook):
    B, T, S = states.shape
    L = word_embeddings.shape[2]
    H = W_state.shape[1]
    D = W2.shape[1]
    CD = W_pi.shape[1]
    K = codebook.shape[0]
    NT = T // _HORIZON

    rows = B * T
    row_tile = 2048
    flat = states.reshape(rows, S)
    ret_state = pl.pallas_call(
        _state_matmul_body,
        grid=(rows // row_tile,),
        in_specs=[
            pl.BlockSpec((row_tile, S), lambda i: (i, 0)),
            pl.BlockSpec((S, H), lambda i: (0, 0)),
            pl.BlockSpec((H,), lambda i: (0,)),
        ],
        out_specs=pl.BlockSpec((row_tile, H), lambda i: (i, 0)),
        out_shape=jax.ShapeDtypeStruct((rows, H), jnp.float32),
    )(flat, W_state, b_state).reshape(B, T, H)

    hs_view = states.reshape(B, NT, _HORIZON * S)
    full = lambda shape: pl.BlockSpec(shape, lambda b: tuple(0 for _ in shape))
    idx3, table, loss11, ent11 = pl.pallas_call(
        _vq_body,
        grid=(B,),
        in_specs=[
            pl.BlockSpec((1, NT, S), lambda b: (b, 0, 0)),
            pl.BlockSpec((1, 1, L), lambda b: (b, 0, 0)),
            full((S, H)), full((H,)),
            full((L, H)), full((H,)),
            full((2 * H, H)), full((H,)),
            full((H, H)), full((H,)),
            full((H, D)), full((D,)),
            full((D, CD)), full((CD,)),
            full((CD, D)), full((D,)),
            full((K, CD)),
        ],
        out_specs=[
            pl.BlockSpec((1, 1, NT), lambda b: (b, 0, 0)),
            full((K, D)),
            full((1, 1)),
            full((1, 1)),
        ],
        out_shape=[
            jax.ShapeDtypeStruct((B, 1, NT), jnp.int32),
            jax.ShapeDtypeStruct((K, D), jnp.float32),
            jax.ShapeDtypeStruct((1, 1), jnp.float32),
            jax.ShapeDtypeStruct((1, 1), jnp.float32),
        ],
    )(hs_view, word_embeddings, W_state, b_state, W_lang, b_lang,
      W0, b0, W1, b1, W2, b2, W_pi, b_pi, W_po, b_po, codebook)

    indices = idx3.reshape(B, NT)

    ntok = B * NT
    n_workers = 32
    b_per_w = ntok // n_workers
    mesh = plsc.VectorSubcoreMesh(core_axis_name="c", subcore_axis_name="s")
    gather = pl.kernel(
        _sc_gather, mesh=mesh,
        out_type=jax.ShapeDtypeStruct((ntok, D), jnp.float32),
        scratch_types=[
            pltpu.VMEM((b_per_w,), jnp.int32),
            pltpu.VMEM((b_per_w, D), jnp.float32),
            pltpu.SemaphoreType.DMA,
        ],
    )
    options = gather(table, indices.reshape(ntok)).reshape(B, NT, D)

    commitment_loss = loss11[0, 0]
    entropies = ent11[0, 0]
    return (options, indices, commitment_loss, entropies, ret_state)


# speedup vs baseline: 1.1832x; 1.0018x over previous
import jax
import jax.numpy as jnp
from jax import lax
from jax.experimental import pallas as pl
from jax.experimental.pallas import tpu as pltpu
from jax.experimental.pallas import tpu_sc as plsc

_HORIZON = 4
_COMMIT_W = 0.25


def _state_matmul_body(x_ref, w_ref, b_ref, o_ref):
    o_ref[...] = (
        jnp.dot(x_ref[...], w_ref[...], preferred_element_type=jnp.float32)
        + b_ref[...][None, :]
    )


def _vq_body(hs_ref, we_ref, w_state_ref, b_state_ref, w_lang_ref, b_lang_ref,
             w0_ref, b0_ref, w1_ref, b1_ref, w2_ref, b2_ref,
             w_pi_ref, b_pi_ref, w_po_ref, b_po_ref, cb_ref,
             idx_ref, table_ref, loss_ref, ent_ref):
    b = pl.program_id(0)
    nt = hs_ref.shape[1]

    @pl.when(b == 0)
    def _init():
        loss_ref[...] = jnp.zeros((1, 1), jnp.float32)
        ent_ref[...] = jnp.zeros((1, 1), jnp.float32)
        table_ref[...] = (
            jnp.dot(cb_ref[...], w_po_ref[...], preferred_element_type=jnp.float32)
            + b_po_ref[...][None, :]
        )

    hs = hs_ref[0]
    se = jnp.dot(hs, w_state_ref[...], preferred_element_type=jnp.float32)
    se = se + b_state_ref[...][None, :]

    we = we_ref[0]
    le = jnp.dot(we, w_lang_ref[...], preferred_element_type=jnp.float32)
    le = le + b_lang_ref[...][None, :]
    le_rep = jnp.broadcast_to(le, (nt, le.shape[1]))

    inp = jnp.concatenate([le_rep, se], axis=-1)
    h = jnp.dot(inp, w0_ref[...], preferred_element_type=jnp.float32) + b0_ref[...][None, :]
    h = jnp.dot(h, w1_ref[...], preferred_element_type=jnp.float32) + b1_ref[...][None, :]
    op = jnp.dot(h, w2_ref[...], preferred_element_type=jnp.float32) + b2_ref[...][None, :]
    x = jnp.dot(op, w_pi_ref[...], preferred_element_type=jnp.float32) + b_pi_ref[...][None, :]

    cb = cb_ref[...]
    xc = lax.dot_general(x, cb, (((1,), (1,)), ((), ())),
                         preferred_element_type=jnp.float32)
    x2 = jnp.sum(x * x, axis=1, keepdims=True)
    c2 = jnp.sum(cb * cb, axis=1)
    d = x2 - 2.0 * xc + c2[None, :]

    idx_ref[0, 0, :] = jnp.argmin(d, axis=1).astype(jnp.int32)

    dmin = jnp.min(d, axis=1)
    cd = w_pi_ref.shape[1]
    loss_scale = _COMMIT_W / (pl.num_programs(0) * nt * cd)
    loss_ref[...] += (jnp.sum(dmin) * loss_scale).reshape(1, 1)

    u = dmin[:, None] - d
    e = jnp.exp(u)
    ones_k = jnp.ones((d.shape[1],), jnp.float32)
    s = jnp.dot(e, ones_k, preferred_element_type=jnp.float32)
    w = jnp.dot(e * u, ones_k, preferred_element_type=jnp.float32)
    ent = jnp.log(s) - w / s
    ent_ref[...] += (jnp.sum(ent) * (1.0 / (pl.num_programs(0) * nt))).reshape(1, 1)


def _sc_gather(table_hbm, idx_hbm, out_hbm, idx_v, rows_v, sem):
    wid = lax.axis_index("s") * 2 + lax.axis_index("c")
    b_per_w = idx_v.shape[0]
    base = wid * b_per_w
    pltpu.sync_copy(idx_hbm.at[pl.ds(base, b_per_w)], idx_v)
    pltpu.async_copy(table_hbm.at[idx_v], rows_v, sem).wait()
    pltpu.sync_copy(rows_v, out_hbm.at[pl.ds(base, b_per_w)])


def kernel(word_embeddings, states, W_state, b_state, W_lang, b_lang,
           W0, b0, W1, b1, W2, b2, W_pi, b_pi, W_po, b_po, codebook):
    B, T, S = states.shape
    L = word_embeddings.shape[2]
    H = W_state.shape[1]
    D = W2.shape[1]
    CD = W_pi.shape[1]
    K = codebook.shape[0]
    NT = T // _HORIZON

    hs_view = states.reshape(B, NT, _HORIZON * S)
    full = lambda shape: pl.BlockSpec(shape, lambda b: tuple(0 for _ in shape))
    idx3, table, loss11, ent11 = pl.pallas_call(
        _vq_body,
        grid=(B,),
        in_specs=[
            pl.BlockSpec((1, NT, S), lambda b: (b, 0, 0)),
            pl.BlockSpec((1, 1, L), lambda b: (b, 0, 0)),
            full((S, H)), full((H,)),
            full((L, H)), full((H,)),
            full((2 * H, H)), full((H,)),
            full((H, H)), full((H,)),
            full((H, D)), full((D,)),
            full((D, CD)), full((CD,)),
            full((CD, D)), full((D,)),
            full((K, CD)),
        ],
        out_specs=[
            pl.BlockSpec((1, 1, NT), lambda b: (b, 0, 0)),
            full((K, D)),
            full((1, 1)),
            full((1, 1)),
        ],
        out_shape=[
            jax.ShapeDtypeStruct((B, 1, NT), jnp.int32),
            jax.ShapeDtypeStruct((K, D), jnp.float32),
            jax.ShapeDtypeStruct((1, 1), jnp.float32),
            jax.ShapeDtypeStruct((1, 1), jnp.float32),
        ],
    )(hs_view, word_embeddings, W_state, b_state, W_lang, b_lang,
      W0, b0, W1, b1, W2, b2, W_pi, b_pi, W_po, b_po, codebook)

    indices = idx3.reshape(B, NT)

    ntok = B * NT
    n_workers = 32
    b_per_w = ntok // n_workers
    mesh = plsc.VectorSubcoreMesh(core_axis_name="c", subcore_axis_name="s")
    gather = pl.kernel(
        _sc_gather, mesh=mesh,
        out_type=jax.ShapeDtypeStruct((ntok, D), jnp.float32),
        scratch_types=[
            pltpu.VMEM((b_per_w,), jnp.int32),
            pltpu.VMEM((b_per_w, D), jnp.float32),
            pltpu.SemaphoreType.DMA,
        ],
    )
    options = gather(table, indices.reshape(ntok)).reshape(B, NT, D)

    rows = B * T
    row_tile = 2048
    flat = states.reshape(rows, S)
    ret_state = pl.pallas_call(
        _state_matmul_body,
        grid=(rows // row_tile,),
        in_specs=[
            pl.BlockSpec((row_tile, S), lambda i: (i, 0)),
            pl.BlockSpec((S, H), lambda i: (0, 0)),
            pl.BlockSpec((H,), lambda i: (0,)),
        ],
        out_specs=pl.BlockSpec((row_tile, H), lambda i: (i, 0)),
        out_shape=jax.ShapeDtypeStruct((rows, H), jnp.float32),
    )(flat, W_state, b_state).reshape(B, T, H)

    commitment_loss = loss11[0, 0]
    entropies = ent11[0, 0]
    return (options, indices, commitment_loss, entropies, ret_state)


# speedup vs baseline: 2.1235x; 1.7947x over previous
import jax
import jax.numpy as jnp
from jax import lax
from jax.experimental import pallas as pl
from jax.experimental.pallas import tpu as pltpu
from jax.experimental.pallas import tpu_sc as plsc

_HORIZON = 4
_COMMIT_W = 0.25


def _fused_body(st_ref, we_ref, w_state_ref, b_state_ref, w_lang_ref, b_lang_ref,
                w0_ref, b0_ref, w1_ref, b1_ref, w2_ref, b2_ref,
                w_pi_ref, b_pi_ref, w_po_ref, b_po_ref, cb_ref,
                ret_ref, idx_ref, table_ref, loss_ref, ent_ref,
                le_ref, c2_ref):
    b = pl.program_id(0)
    nb = pl.num_programs(0)

    @pl.when(b == 0)
    def _init():
        loss_ref[...] = jnp.zeros((1, 1), jnp.float32)
        ent_ref[...] = jnp.zeros((1, 1), jnp.float32)
        cb = cb_ref[...]
        table_ref[...] = (
            jnp.dot(cb, w_po_ref[...], preferred_element_type=jnp.float32)
            + b_po_ref[...][None, :]
        )
        c2_ref[...] = jnp.sum(cb * cb, axis=1)[None, :]
        le_ref[...] = (
            jnp.dot(we_ref[:, 0, :], w_lang_ref[...],
                    preferred_element_type=jnp.float32)
            + b_lang_ref[...][None, :]
        )

    x_all = st_ref[0]
    rs = jnp.dot(x_all, w_state_ref[...], preferred_element_type=jnp.float32)
    rs = rs + b_state_ref[...][None, :]
    ret_ref[0] = rs

    nt = rs.shape[0] // _HORIZON
    se = rs.reshape(nt, _HORIZON, rs.shape[1])[:, 0, :]

    le = le_ref[pl.ds(b, 1), :]
    le_rep = jnp.broadcast_to(le, (nt, le.shape[1]))
    inp = jnp.concatenate([le_rep, se], axis=-1)
    h = jnp.dot(inp, w0_ref[...], preferred_element_type=jnp.float32) + b0_ref[...][None, :]
    h = jnp.dot(h, w1_ref[...], preferred_element_type=jnp.float32) + b1_ref[...][None, :]
    op = jnp.dot(h, w2_ref[...], preferred_element_type=jnp.float32) + b2_ref[...][None, :]
    x = jnp.dot(op, w_pi_ref[...], preferred_element_type=jnp.float32) + b_pi_ref[...][None, :]

    xc = lax.dot_general(x, cb_ref[...], (((1,), (1,)), ((), ())),
                         preferred_element_type=jnp.float32)
    x2 = jnp.sum(x * x, axis=1, keepdims=True)
    d = x2 - 2.0 * xc + c2_ref[...]

    dmin = jnp.min(d, axis=1)
    k = d.shape[1]
    iota = lax.broadcasted_iota(jnp.int32, d.shape, 1)
    hit = jnp.where(d == dmin[:, None], iota, k)
    idx_ref[0, 0, :] = jnp.min(hit, axis=1)

    cd = w_pi_ref.shape[1]
    loss_scale = _COMMIT_W / (nb * nt * cd)
    loss_ref[...] += (jnp.sum(dmin) * loss_scale).reshape(1, 1)

    u = dmin[:, None] - d
    e = jnp.exp(u)
    ones_k = jnp.ones((k,), jnp.float32)
    s = jnp.dot(e, ones_k, preferred_element_type=jnp.float32)
    w = jnp.dot(e * u, ones_k, preferred_element_type=jnp.float32)
    ent = jnp.log(s) - w / s
    ent_ref[...] += (jnp.sum(ent) * (1.0 / (nb * nt))).reshape(1, 1)


def _sc_gather(table_hbm, idx_hbm, out_hbm, idx_v, rows_v, sem):
    wid = lax.axis_index("s") * 2 + lax.axis_index("c")
    b_per_w = idx_v.shape[0]
    base = wid * b_per_w
    pltpu.sync_copy(idx_hbm.at[pl.ds(base, b_per_w)], idx_v)
    nchunk = 8
    csz = b_per_w // nchunk
    copies = [
        pltpu.async_copy(table_hbm.at[idx_v.at[pl.ds(j * csz, csz)]],
                         rows_v.at[pl.ds(j * csz, csz)], sem)
        for j in range(nchunk)
    ]
    for c in copies:
        c.wait()
    pltpu.sync_copy(rows_v, out_hbm.at[pl.ds(base, b_per_w)])


def kernel(word_embeddings, states, W_state, b_state, W_lang, b_lang,
           W0, b0, W1, b1, W2, b2, W_pi, b_pi, W_po, b_po, codebook):
    B, T, S = states.shape
    L = word_embeddings.shape[2]
    H = W_state.shape[1]
    D = W2.shape[1]
    CD = W_pi.shape[1]
    K = codebook.shape[0]
    NT = T // _HORIZON

    full = lambda shape: pl.BlockSpec(shape, lambda b: tuple(0 for _ in shape))
    ret3, idx3, table, loss11, ent11 = pl.pallas_call(
        _fused_body,
        grid=(B,),
        in_specs=[
            pl.BlockSpec((1, T, S), lambda b: (b, 0, 0)),
            full((B, 1, L)),
            full((S, H)), full((H,)),
            full((L, H)), full((H,)),
            full((2 * H, H)), full((H,)),
            full((H, H)), full((H,)),
            full((H, D)), full((D,)),
            full((D, CD)), full((CD,)),
            full((CD, D)), full((D,)),
            full((K, CD)),
        ],
        out_specs=[
            pl.BlockSpec((1, T, H), lambda b: (b, 0, 0)),
            pl.BlockSpec((1, 1, NT), lambda b: (b, 0, 0)),
            full((K, D)),
            full((1, 1)),
            full((1, 1)),
        ],
        out_shape=[
            jax.ShapeDtypeStruct((B, T, H), jnp.float32),
            jax.ShapeDtypeStruct((B, 1, NT), jnp.int32),
            jax.ShapeDtypeStruct((K, D), jnp.float32),
            jax.ShapeDtypeStruct((1, 1), jnp.float32),
            jax.ShapeDtypeStruct((1, 1), jnp.float32),
        ],
        scratch_shapes=[
            pltpu.VMEM((B, H), jnp.float32),
            pltpu.VMEM((1, K), jnp.float32),
        ],
    )(states, word_embeddings, W_state, b_state, W_lang, b_lang,
      W0, b0, W1, b1, W2, b2, W_pi, b_pi, W_po, b_po, codebook)

    indices = idx3.reshape(B, NT)

    ntok = B * NT
    n_workers = 32
    b_per_w = ntok // n_workers
    mesh = plsc.VectorSubcoreMesh(core_axis_name="c", subcore_axis_name="s")
    gather = pl.kernel(
        _sc_gather, mesh=mesh,
        out_type=jax.ShapeDtypeStruct((ntok, D), jnp.float32),
        scratch_types=[
            pltpu.VMEM((b_per_w,), jnp.int32),
            pltpu.VMEM((b_per_w, D), jnp.float32),
            pltpu.SemaphoreType.DMA,
        ],
    )
    options = gather(table, indices.reshape(ntok)).reshape(B, NT, D)

    commitment_loss = loss11[0, 0]
    entropies = ent11[0, 0]
    return (options, indices, commitment_loss, entropies, ret3)


# speedup vs baseline: 2.1257x; 1.0010x over previous
import jax
import jax.numpy as jnp
from jax import lax
from jax.experimental import pallas as pl
from jax.experimental.pallas import tpu as pltpu
from jax.experimental.pallas import tpu_sc as plsc

_HORIZON = 4
_COMMIT_W = 0.25


def _fused_body(st_ref, we_ref, w_state_ref, b_state_ref, w_lang_ref, b_lang_ref,
                w0_ref, b0_ref, w1_ref, b1_ref, w2_ref, b2_ref,
                w_pi_ref, b_pi_ref, w_po_ref, b_po_ref, cb_ref,
                ret_ref, idx_ref, table_ref, loss_ref, ent_ref,
                le_ref, c2_ref):
    b = pl.program_id(0)
    nb = pl.num_programs(0)

    @pl.when(b == 0)
    def _init():
        loss_ref[...] = jnp.zeros((1, 1), jnp.float32)
        ent_ref[...] = jnp.zeros((1, 1), jnp.float32)
        cb = cb_ref[...]
        table_ref[...] = (
            jnp.dot(cb, w_po_ref[...], preferred_element_type=jnp.float32)
            + b_po_ref[...][None, :]
        )
        c2_ref[...] = jnp.sum(cb * cb, axis=1)[None, :]
        le_ref[...] = (
            jnp.dot(we_ref[:, 0, :], w_lang_ref[...],
                    preferred_element_type=jnp.float32)
            + b_lang_ref[...][None, :]
        )

    x_all = st_ref[0]
    rs = jnp.dot(x_all, w_state_ref[...], preferred_element_type=jnp.float32)
    rs = rs + b_state_ref[...][None, :]
    ret_ref[0] = rs

    nt = rs.shape[0] // _HORIZON
    se = rs.reshape(nt, _HORIZON, rs.shape[1])[:, 0, :]

    le = le_ref[pl.ds(b, 1), :]
    le_rep = jnp.broadcast_to(le, (nt, le.shape[1]))
    inp = jnp.concatenate([le_rep, se], axis=-1)
    h = jnp.dot(inp, w0_ref[...], preferred_element_type=jnp.float32) + b0_ref[...][None, :]
    h = jnp.dot(h, w1_ref[...], preferred_element_type=jnp.float32) + b1_ref[...][None, :]
    op = jnp.dot(h, w2_ref[...], preferred_element_type=jnp.float32) + b2_ref[...][None, :]
    x = jnp.dot(op, w_pi_ref[...], preferred_element_type=jnp.float32) + b_pi_ref[...][None, :]

    xc = lax.dot_general(x, cb_ref[...], (((1,), (1,)), ((), ())),
                         preferred_element_type=jnp.float32)
    x2 = jnp.sum(x * x, axis=1, keepdims=True)
    d = x2 - 2.0 * xc + c2_ref[...]

    dmin = jnp.min(d, axis=1)
    k = d.shape[1]
    iota = lax.broadcasted_iota(jnp.int32, d.shape, 1)
    hit = jnp.where(d == dmin[:, None], iota, k)
    idx_ref[0, 0, :] = jnp.min(hit, axis=1)

    cd = w_pi_ref.shape[1]
    loss_scale = _COMMIT_W / (nb * nt * cd)
    loss_ref[...] += (jnp.sum(dmin) * loss_scale).reshape(1, 1)

    u = dmin[:, None] - d
    e = jnp.exp(u)
    ones_k = jnp.ones((k,), jnp.float32)
    s = jnp.dot(e, ones_k, preferred_element_type=jnp.float32)
    w = jnp.dot(e * u, ones_k, preferred_element_type=jnp.float32)
    ent = jnp.log(s) - w / s
    ent_ref[...] += (jnp.sum(ent) * (1.0 / (nb * nt))).reshape(1, 1)


def _sc_gather(table_hbm, idx_hbm, out_hbm, idx_v, rows_v, gsem, wsem):
    wid = lax.axis_index("s") * 2 + lax.axis_index("c")
    b_per_w = idx_v.shape[0]
    base = wid * b_per_w
    half = b_per_w // 2
    pltpu.sync_copy(idx_hbm.at[pl.ds(base, b_per_w)], idx_v)
    nchunk = 16
    csz = b_per_w // nchunk
    copies = [
        pltpu.async_copy(table_hbm.at[idx_v.at[pl.ds(j * csz, csz)]],
                         rows_v.at[pl.ds(j * csz, csz)], gsem)
        for j in range(nchunk)
    ]
    for c in copies[: nchunk // 2]:
        c.wait()
    w0 = pltpu.async_copy(rows_v.at[pl.ds(0, half)],
                          out_hbm.at[pl.ds(base, half)], wsem)
    for c in copies[nchunk // 2:]:
        c.wait()
    w1 = pltpu.async_copy(rows_v.at[pl.ds(half, half)],
                          out_hbm.at[pl.ds(base + half, half)], wsem)
    w0.wait()
    w1.wait()


def kernel(word_embeddings, states, W_state, b_state, W_lang, b_lang,
           W0, b0, W1, b1, W2, b2, W_pi, b_pi, W_po, b_po, codebook):
    B, T, S = states.shape
    L = word_embeddings.shape[2]
    H = W_state.shape[1]
    D = W2.shape[1]
    CD = W_pi.shape[1]
    K = codebook.shape[0]
    NT = T // _HORIZON

    full = lambda shape: pl.BlockSpec(shape, lambda b: tuple(0 for _ in shape))
    ret3, idx3, table, loss11, ent11 = pl.pallas_call(
        _fused_body,
        grid=(B,),
        in_specs=[
            pl.BlockSpec((1, T, S), lambda b: (b, 0, 0)),
            full((B, 1, L)),
            full((S, H)), full((H,)),
            full((L, H)), full((H,)),
            full((2 * H, H)), full((H,)),
            full((H, H)), full((H,)),
            full((H, D)), full((D,)),
            full((D, CD)), full((CD,)),
            full((CD, D)), full((D,)),
            full((K, CD)),
        ],
        out_specs=[
            pl.BlockSpec((1, T, H), lambda b: (b, 0, 0)),
            pl.BlockSpec((1, 1, NT), lambda b: (b, 0, 0)),
            full((K, D)),
            full((1, 1)),
            full((1, 1)),
        ],
        out_shape=[
            jax.ShapeDtypeStruct((B, T, H), jnp.float32),
            jax.ShapeDtypeStruct((B, 1, NT), jnp.int32),
            jax.ShapeDtypeStruct((K, D), jnp.float32),
            jax.ShapeDtypeStruct((1, 1), jnp.float32),
            jax.ShapeDtypeStruct((1, 1), jnp.float32),
        ],
        scratch_shapes=[
            pltpu.VMEM((B, H), jnp.float32),
            pltpu.VMEM((1, K), jnp.float32),
        ],
    )(states, word_embeddings, W_state, b_state, W_lang, b_lang,
      W0, b0, W1, b1, W2, b2, W_pi, b_pi, W_po, b_po, codebook)

    indices = idx3.reshape(B, NT)

    ntok = B * NT
    n_workers = 32
    b_per_w = ntok // n_workers
    mesh = plsc.VectorSubcoreMesh(core_axis_name="c", subcore_axis_name="s")
    gather = pl.kernel(
        _sc_gather, mesh=mesh,
        out_type=jax.ShapeDtypeStruct((ntok, D), jnp.float32),
        scratch_types=[
            pltpu.VMEM((b_per_w,), jnp.int32),
            pltpu.VMEM((b_per_w, D), jnp.float32),
            pltpu.SemaphoreType.DMA,
            pltpu.SemaphoreType.DMA,
        ],
    )
    options = gather(table, indices.reshape(ntok)).reshape(B, NT, D)

    commitment_loss = loss11[0, 0]
    entropies = ent11[0, 0]
    return (options, indices, commitment_loss, entropies, ret3)


# speedup vs baseline: 2.2680x; 1.0669x over previous
import jax
import jax.numpy as jnp
from jax import lax
from jax.experimental import pallas as pl
from jax.experimental.pallas import tpu as pltpu
from jax.experimental.pallas import tpu_sc as plsc

_HORIZON = 4
_COMMIT_W = 0.25


def _fused_body(st_ref, we_ref, w_state_ref, b_state_ref, w_lang_ref, b_lang_ref,
                w0_ref, b0_ref, w1_ref, b1_ref, w2_ref, b2_ref,
                w_pi_ref, b_pi_ref, w_po_ref, b_po_ref, cb_ref,
                ret_ref, idx_ref, table_ref, loss_ref, ent_ref,
                le_ref, c2_ref):
    b = pl.program_id(0)
    nb = pl.num_programs(0)

    @pl.when(b == 0)
    def _init():
        loss_ref[...] = jnp.zeros((1, 1), jnp.float32)
        ent_ref[...] = jnp.zeros((1, 1), jnp.float32)
        cb = cb_ref[...]
        table_ref[...] = (
            jnp.dot(cb, w_po_ref[...], preferred_element_type=jnp.float32)
            + b_po_ref[...][None, :]
        )
        c2_ref[...] = jnp.sum(cb * cb, axis=1)[None, :]
        le_ref[...] = (
            jnp.dot(we_ref[:, 0, :], w_lang_ref[...],
                    preferred_element_type=jnp.float32)
            + b_lang_ref[...][None, :]
        )

    x_all = st_ref[0]
    rs = jnp.dot(x_all, w_state_ref[...], preferred_element_type=jnp.float32)
    rs = rs + b_state_ref[...][None, :]
    ret_ref[0] = rs

    nt = x_all.shape[0] // _HORIZON
    hs = x_all.reshape(nt, _HORIZON, x_all.shape[1])[:, 0, :]
    se = jnp.dot(hs, w_state_ref[...], preferred_element_type=jnp.float32)
    se = se + b_state_ref[...][None, :]

    le = le_ref[pl.ds(b, 1), :]
    le_rep = jnp.broadcast_to(le, (nt, le.shape[1]))
    inp = jnp.concatenate([le_rep, se], axis=-1)
    h = jnp.dot(inp, w0_ref[...], preferred_element_type=jnp.float32) + b0_ref[...][None, :]
    h = jnp.dot(h, w1_ref[...], preferred_element_type=jnp.float32) + b1_ref[...][None, :]
    op = jnp.dot(h, w2_ref[...], preferred_element_type=jnp.float32) + b2_ref[...][None, :]
    x = jnp.dot(op, w_pi_ref[...], preferred_element_type=jnp.float32) + b_pi_ref[...][None, :]

    xc = lax.dot_general(x, cb_ref[...], (((1,), (1,)), ((), ())),
                         preferred_element_type=jnp.float32)
    x2 = jnp.sum(x * x, axis=1, keepdims=True)
    d = x2 - 2.0 * xc + c2_ref[...]

    dmin = jnp.min(d, axis=1)
    k = d.shape[1]
    iota = lax.broadcasted_iota(jnp.int32, d.shape, 1)
    hit = jnp.where(d == dmin[:, None], iota, k)
    idx_ref[0, 0, :] = jnp.min(hit, axis=1)

    cd = w_pi_ref.shape[1]
    loss_scale = _COMMIT_W / (nb * nt * cd)
    loss_ref[...] += (jnp.sum(dmin) * loss_scale).reshape(1, 1)

    u = dmin[:, None] - d
    e = jnp.exp(u)
    ones_k = jnp.ones((k,), jnp.float32)
    s = jnp.dot(e, ones_k, preferred_element_type=jnp.float32)
    w = jnp.dot(e * u, ones_k, preferred_element_type=jnp.float32)
    ent = jnp.log(s) - w / s
    ent_ref[...] += (jnp.sum(ent) * (1.0 / (nb * nt))).reshape(1, 1)


def _sc_gather(table_hbm, idx_hbm, out_hbm, idx_v, rows_v, gsem, wsem):
    wid = lax.axis_index("s") * 2 + lax.axis_index("c")
    b_per_w = idx_v.shape[0]
    base = wid * b_per_w
    half = b_per_w // 2
    pltpu.sync_copy(idx_hbm.at[pl.ds(base, b_per_w)], idx_v)
    nchunk = 16
    csz = b_per_w // nchunk
    copies = [
        pltpu.async_copy(table_hbm.at[idx_v.at[pl.ds(j * csz, csz)]],
                         rows_v.at[pl.ds(j * csz, csz)], gsem)
        for j in range(nchunk)
    ]
    for c in copies[: nchunk // 2]:
        c.wait()
    w0 = pltpu.async_copy(rows_v.at[pl.ds(0, half)],
                          out_hbm.at[pl.ds(base, half)], wsem)
    for c in copies[nchunk // 2:]:
        c.wait()
    w1 = pltpu.async_copy(rows_v.at[pl.ds(half, half)],
                          out_hbm.at[pl.ds(base + half, half)], wsem)
    w0.wait()
    w1.wait()


def kernel(word_embeddings, states, W_state, b_state, W_lang, b_lang,
           W0, b0, W1, b1, W2, b2, W_pi, b_pi, W_po, b_po, codebook):
    B, T, S = states.shape
    L = word_embeddings.shape[2]
    H = W_state.shape[1]
    D = W2.shape[1]
    CD = W_pi.shape[1]
    K = codebook.shape[0]
    NT = T // _HORIZON

    full = lambda shape: pl.BlockSpec(shape, lambda b: tuple(0 for _ in shape))
    ret3, idx3, table, loss11, ent11 = pl.pallas_call(
        _fused_body,
        grid=(B,),
        in_specs=[
            pl.BlockSpec((1, T, S), lambda b: (b, 0, 0)),
            full((B, 1, L)),
            full((S, H)), full((H,)),
            full((L, H)), full((H,)),
            full((2 * H, H)), full((H,)),
            full((H, H)), full((H,)),
            full((H, D)), full((D,)),
            full((D, CD)), full((CD,)),
            full((CD, D)), full((D,)),
            full((K, CD)),
        ],
        out_specs=[
            pl.BlockSpec((1, T, H), lambda b: (b, 0, 0)),
            pl.BlockSpec((1, 1, NT), lambda b: (b, 0, 0)),
            full((K, D)),
            full((1, 1)),
            full((1, 1)),
        ],
        out_shape=[
            jax.ShapeDtypeStruct((B, T, H), jnp.float32),
            jax.ShapeDtypeStruct((B, 1, NT), jnp.int32),
            jax.ShapeDtypeStruct((K, D), jnp.float32),
            jax.ShapeDtypeStruct((1, 1), jnp.float32),
            jax.ShapeDtypeStruct((1, 1), jnp.float32),
        ],
        scratch_shapes=[
            pltpu.VMEM((B, H), jnp.float32),
            pltpu.VMEM((1, K), jnp.float32),
        ],
    )(states, word_embeddings, W_state, b_state, W_lang, b_lang,
      W0, b0, W1, b1, W2, b2, W_pi, b_pi, W_po, b_po, codebook)

    indices = idx3.reshape(B, NT)

    ntok = B * NT
    n_workers = 32
    b_per_w = ntok // n_workers
    mesh = plsc.VectorSubcoreMesh(core_axis_name="c", subcore_axis_name="s")
    gather = pl.kernel(
        _sc_gather, mesh=mesh,
        out_type=jax.ShapeDtypeStruct((ntok, D), jnp.float32),
        scratch_types=[
            pltpu.VMEM((b_per_w,), jnp.int32),
            pltpu.VMEM((b_per_w, D), jnp.float32),
            pltpu.SemaphoreType.DMA,
            pltpu.SemaphoreType.DMA,
        ],
    )
    options = gather(table, indices.reshape(ntok)).reshape(B, NT, D)

    commitment_loss = loss11[0, 0]
    entropies = ent11[0, 0]
    return (options, indices, commitment_loss, entropies, ret3)


# speedup vs baseline: 2.4895x; 1.0977x over previous
import jax
import jax.numpy as jnp
from jax import lax
from jax.experimental import pallas as pl
from jax.experimental.pallas import tpu as pltpu
from jax.experimental.pallas import tpu_sc as plsc

_HORIZON = 4
_COMMIT_W = 0.25
_TC_BATCHES = 12


def _fused_body(st_ref, we_ref, w_state_ref, b_state_ref, w_lang_ref, b_lang_ref,
                w0_ref, b0_ref, w1_ref, b1_ref, w2_ref, b2_ref,
                w_pi_ref, b_pi_ref, w_po_ref, b_po_ref, cb_ref,
                ret_ref, idx_ref, table_ref, loss_ref, ent_ref, opt_ref,
                le_ref, c2_ref):
    b = pl.program_id(0)
    nb = pl.num_programs(0)

    @pl.when(b == 0)
    def _init():
        loss_ref[...] = jnp.zeros((1, 1), jnp.float32)
        ent_ref[...] = jnp.zeros((1, 1), jnp.float32)
        cb = cb_ref[...]
        table_ref[...] = (
            jnp.dot(cb, w_po_ref[...], preferred_element_type=jnp.float32)
            + b_po_ref[...][None, :]
        )
        c2_ref[...] = jnp.sum(cb * cb, axis=1)[None, :]
        le_ref[...] = (
            jnp.dot(we_ref[:, 0, :], w_lang_ref[...],
                    preferred_element_type=jnp.float32)
            + b_lang_ref[...][None, :]
        )

    x_all = st_ref[0]
    rs = jnp.dot(x_all, w_state_ref[...], preferred_element_type=jnp.float32)
    rs = rs + b_state_ref[...][None, :]
    ret_ref[0] = rs

    nt = x_all.shape[0] // _HORIZON
    hs = x_all.reshape(nt, _HORIZON, x_all.shape[1])[:, 0, :]
    se = jnp.dot(hs, w_state_ref[...], preferred_element_type=jnp.float32)
    se = se + b_state_ref[...][None, :]

    le = le_ref[pl.ds(b, 1), :]
    le_rep = jnp.broadcast_to(le, (nt, le.shape[1]))
    inp = jnp.concatenate([le_rep, se], axis=-1)
    h = jnp.dot(inp, w0_ref[...], preferred_element_type=jnp.float32) + b0_ref[...][None, :]
    h = jnp.dot(h, w1_ref[...], preferred_element_type=jnp.float32) + b1_ref[...][None, :]
    op = jnp.dot(h, w2_ref[...], preferred_element_type=jnp.float32) + b2_ref[...][None, :]
    x = jnp.dot(op, w_pi_ref[...], preferred_element_type=jnp.float32) + b_pi_ref[...][None, :]

    xc = lax.dot_general(x, cb_ref[...], (((1,), (1,)), ((), ())),
                         preferred_element_type=jnp.float32)
    x2 = jnp.sum(x * x, axis=1, keepdims=True)
    d = x2 - 2.0 * xc + c2_ref[...]

    dmin = jnp.min(d, axis=1)
    k = d.shape[1]
    iota = lax.broadcasted_iota(jnp.int32, d.shape, 1)
    hit = jnp.where(d == dmin[:, None], iota, k)
    idx = jnp.min(hit, axis=1)
    idx_ref[0, 0, :] = idx

    @pl.when(b < _TC_BATCHES)
    def _opt_inline():
        onehot = jnp.where(iota == idx[:, None], 1.0, 0.0).astype(jnp.float32)
        opt_ref[0] = jnp.dot(onehot, table_ref[...],
                             preferred_element_type=jnp.float32)

    cd = w_pi_ref.shape[1]
    loss_scale = _COMMIT_W / (nb * nt * cd)
    loss_ref[...] += (jnp.sum(dmin) * loss_scale).reshape(1, 1)

    u = dmin[:, None] - d
    e = jnp.exp(u)
    ones_k = jnp.ones((k,), jnp.float32)
    s = jnp.dot(e, ones_k, preferred_element_type=jnp.float32)
    w = jnp.dot(e * u, ones_k, preferred_element_type=jnp.float32)
    ent = jnp.log(s) - w / s
    ent_ref[...] += (jnp.sum(ent) * (1.0 / (nb * nt))).reshape(1, 1)


def _sc_gather(table_hbm, idx_hbm, opt_tc_hbm, out_hbm, idx_v, rows_v, pt_v,
               gsem, wsem):
    wid = lax.axis_index("s") * 2 + lax.axis_index("c")
    b_per_w = idx_v.shape[0]
    base = idx_hbm.shape[0] - (32 - wid) * b_per_w
    pltpu.sync_copy(idx_hbm.at[pl.ds(base, b_per_w)], idx_v)
    nchunk = 4
    csz = b_per_w // nchunk
    copies = [
        pltpu.async_copy(table_hbm.at[idx_v.at[pl.ds(j * csz, csz)]],
                         rows_v.at[pl.ds(j * csz, csz)], gsem)
        for j in range(nchunk)
    ]
    n_pt = pt_v.shape[0]
    pbase = wid * n_pt
    pltpu.sync_copy(opt_tc_hbm.at[pl.ds(pbase, n_pt)], pt_v)
    w_pt = pltpu.async_copy(pt_v, out_hbm.at[pl.ds(pbase, n_pt)], wsem)
    for c in copies:
        c.wait()
    w_g = pltpu.async_copy(rows_v, out_hbm.at[pl.ds(base, b_per_w)], wsem)
    w_pt.wait()
    w_g.wait()


def kernel(word_embeddings, states, W_state, b_state, W_lang, b_lang,
           W0, b0, W1, b1, W2, b2, W_pi, b_pi, W_po, b_po, codebook):
    B, T, S = states.shape
    L = word_embeddings.shape[2]
    H = W_state.shape[1]
    D = W2.shape[1]
    CD = W_pi.shape[1]
    K = codebook.shape[0]
    NT = T // _HORIZON

    full = lambda shape: pl.BlockSpec(shape, lambda b: tuple(0 for _ in shape))
    ret3, idx3, table, loss11, ent11, opt3 = pl.pallas_call(
        _fused_body,
        grid=(B,),
        in_specs=[
            pl.BlockSpec((1, T, S), lambda b: (b, 0, 0)),
            full((B, 1, L)),
            full((S, H)), full((H,)),
            full((L, H)), full((H,)),
            full((2 * H, H)), full((H,)),
            full((H, H)), full((H,)),
            full((H, D)), full((D,)),
            full((D, CD)), full((CD,)),
            full((CD, D)), full((D,)),
            full((K, CD)),
        ],
        out_specs=[
            pl.BlockSpec((1, T, H), lambda b: (b, 0, 0)),
            pl.BlockSpec((1, 1, NT), lambda b: (b, 0, 0)),
            full((K, D)),
            full((1, 1)),
            full((1, 1)),
            pl.BlockSpec((1, NT, D),
                         lambda b: (jnp.minimum(b, _TC_BATCHES - 1), 0, 0)),
        ],
        out_shape=[
            jax.ShapeDtypeStruct((B, T, H), jnp.float32),
            jax.ShapeDtypeStruct((B, 1, NT), jnp.int32),
            jax.ShapeDtypeStruct((K, D), jnp.float32),
            jax.ShapeDtypeStruct((1, 1), jnp.float32),
            jax.ShapeDtypeStruct((1, 1), jnp.float32),
            jax.ShapeDtypeStruct((_TC_BATCHES, NT, D), jnp.float32),
        ],
        scratch_shapes=[
            pltpu.VMEM((B, H), jnp.float32),
            pltpu.VMEM((1, K), jnp.float32),
        ],
    )(states, word_embeddings, W_state, b_state, W_lang, b_lang,
      W0, b0, W1, b1, W2, b2, W_pi, b_pi, W_po, b_po, codebook)

    indices = idx3.reshape(B, NT)

    ntok = B * NT
    n_workers = 32
    b_per_w = (B - _TC_BATCHES) * NT // n_workers
    n_pt = _TC_BATCHES * NT // n_workers
    mesh = plsc.VectorSubcoreMesh(core_axis_name="c", subcore_axis_name="s")
    gather = pl.kernel(
        _sc_gather, mesh=mesh,
        out_type=jax.ShapeDtypeStruct((ntok, D), jnp.float32),
        scratch_types=[
            pltpu.VMEM((b_per_w,), jnp.int32),
            pltpu.VMEM((b_per_w, D), jnp.float32),
            pltpu.VMEM((n_pt, D), jnp.float32),
            pltpu.SemaphoreType.DMA,
            pltpu.SemaphoreType.DMA,
        ],
    )
    options = gather(table, indices.reshape(ntok),
                     opt3.reshape(_TC_BATCHES * NT, D)).reshape(B, NT, D)

    commitment_loss = loss11[0, 0]
    entropies = ent11[0, 0]
    return (options, indices, commitment_loss, entropies, ret3)
